# baseline (speedup 1.0000x reference)
import functools

import jax
import jax.numpy as jnp
from jax import lax
from jax.experimental import pallas as pl
from jax.experimental.pallas import tpu as pltpu
from jax.experimental.pallas import tpu_sc as plsc

L = 12
SRC = 2048 + 2
TGT = 2048 + 2
D = 1024

_TBL = L * SRC * D
_NW = 32
_PW = _TBL // _NW
_NCH = 20
_CH = _PW // _NCH
_NBUF = 3
_TOTAL_CHUNKS = 3 * _NCH


def _sc_copy(enc_hbm, self_hbm, cross_hbm, out_hbm,
             buf0, buf1, buf2, rs0, rs1, rs2, ws0, ws1, ws2):
    nc = plsc.get_sparse_core_info().num_cores
    wid = lax.axis_index("s") * nc + lax.axis_index("c")
    base = wid * _PW
    srcs = (enc_hbm, self_hbm, cross_hbm)
    bufs = (buf0, buf1, buf2)
    rsems = (rs0, rs1, rs2)
    wsems = (ws0, ws1, ws2)

    def rd(k):
        t, c = divmod(k, _NCH)
        b = k % _NBUF
        src = srcs[t].at[pl.ds(base + c * _CH, _CH)]
        return pltpu.make_async_copy(src, bufs[b], rsems[b])

    def wr(k):
        t, c = divmod(k, _NCH)
        b = k % _NBUF
        dst = out_hbm.at[pl.ds(t * _TBL + base + c * _CH, _CH)]
        return pltpu.make_async_copy(bufs[b], dst, wsems[b])

    rd(0).start()
    for k in range(_TOTAL_CHUNKS):
        if k + 1 < _TOTAL_CHUNKS:
            if k >= 2:
                wr(k - 2).wait()
            rd(k + 1).start()
        rd(k).wait()
        wr(k).start()
    wr(_TOTAL_CHUNKS - 2).wait()
    wr(_TOTAL_CHUNKS - 1).wait()


def kernel(bsz, enc_w, self_w, cross_w):
    del bsz
    enc2 = enc_w.reshape(_TBL)
    self2 = self_w.reshape(_TBL)
    cross2 = cross_w.reshape(_TBL)
    mesh = plsc.VectorSubcoreMesh(core_axis_name="c", subcore_axis_name="s")
    run = pl.kernel(
        _sc_copy,
        out_type=jax.ShapeDtypeStruct((3 * _TBL,), jnp.float32),
        mesh=mesh,
        scratch_types=(
            [pltpu.VMEM((_CH,), jnp.float32)] * _NBUF
            + [pltpu.SemaphoreType.DMA] * (2 * _NBUF)
        ),
    )
    out = run(enc2, self2, cross2)
    return out.reshape(3, L, SRC, D)
